# Initial kernel scaffold; baseline (speedup 1.0000x reference)
import jax
import jax.numpy as jnp
from jax.experimental import pallas as pl


def kernel(features, edge_index, W, b):
    raise NotImplementedError("write your pallas kernel here")


# speedup vs baseline: 4.7444x; 4.7444x over previous
import functools

import jax
import jax.numpy as jnp
from jax import lax
from jax.experimental import pallas as pl
from jax.experimental.pallas import tpu as pltpu
from jax.experimental.pallas import tpu_sc as plsc

N = 10000
E = 320000
D = 128
NC = 2
NS = 16
HALF = N // NC
DEG_ROWS = 5120
TRASH = HALF
BLK = 128
NBLK = E // BLK
HOP_BPC = NBLK // NC

_MESH = plsc.VectorSubcoreMesh(core_axis_name="c", subcore_axis_name="s")
_SC_PARAMS = pltpu.CompilerParams(needs_layout_passes=False)


def _newton_rsqrt(d):
    i = lax.bitcast_convert_type(d, jnp.int32)
    i = jnp.int32(0x5F3759DF) - (i >> 1)
    y = lax.bitcast_convert_type(i, jnp.float32)
    for _ in range(3):
        y = y * (1.5 - 0.5 * d * y * y)
    return y


def _deg_body(feat_hbm, dst_hbm, norm_hbm, norm2_hbm, t1_hbm,
              deg_sh, dbuf, dloc, ones_b, zbuf, dcopy, nbuf, n2buf,
              hbuf, tbuf):
    c = lax.axis_index("c")
    s = lax.axis_index("s")
    lo = c * HALF

    zero16 = jnp.zeros((16,), jnp.float32)
    for i in range(20):
        zbuf[pl.ds(i * 16, 16)] = zero16
    one16 = jnp.ones((16,), jnp.float32)
    for i in range(8):
        ones_b[pl.ds(i * 16, 16)] = one16

    pltpu.sync_copy(zbuf, deg_sh.at[pl.ds(s * 320, 320)])
    plsc.subcore_barrier()

    nb = jnp.where(s < NBLK % NS, NBLK // NS + 1, NBLK // NS)

    def blk_body(j, _):
        g = s + NS * j
        pltpu.sync_copy(dst_hbm.at[pl.ds(g * BLK, BLK)], dbuf)
        for i in range(BLK // 16):
            d = dbuf[pl.ds(i * 16, 16)]
            inh = jnp.logical_and(d >= lo, d < lo + HALF)
            dloc[0, pl.ds(i * 16, 16)] = jnp.where(inh, d - lo,
                                                   jnp.int32(TRASH))
        pltpu.sync_copy(ones_b, deg_sh.at[dloc.at[0]], add=True)
        return _

    lax.fori_loop(0, nb, blk_body, 0, unroll=False)
    plsc.subcore_barrier()

    NCHUNK = (HALF + 15) // 16

    def chunk_body(j, _):
        k = s + NS * j

        @pl.when(k < NCHUNK)
        def _do():
            base = jnp.minimum(k * 16, HALF - 16)
            pltpu.sync_copy(deg_sh.at[pl.ds(base, 16)], dcopy)
            d = jnp.maximum(dcopy[...], 1.0)
            nrm = _newton_rsqrt(d)
            nbuf[...] = nrm
            n2buf[...] = 1.0 / d
            gbase = lo + base
            pltpu.sync_copy(nbuf, norm_hbm.at[pl.ds(gbase, 16)])
            pltpu.sync_copy(n2buf, norm2_hbm.at[pl.ds(gbase, 16)])
            pltpu.sync_copy(feat_hbm.at[pl.ds(gbase, 16)], hbuf)

            def row_body(r, _):
                sr = plsc.load_gather(nbuf, [jnp.full((16,), r, jnp.int32)])
                for q in range(D // 16):
                    tbuf[r, pl.ds(q * 16, 16)] = (
                        hbuf[r, pl.ds(q * 16, 16)] * sr)
                return _

            lax.fori_loop(0, 16, row_body, 0, unroll=False)
            pltpu.sync_copy(tbuf, t1_hbm.at[pl.ds(gbase, 16)])

        return _

    lax.fori_loop(0, (NCHUNK + NS - 1) // NS, chunk_body, 0, unroll=False)


@functools.partial(
    pl.kernel,
    out_type=(
        jax.ShapeDtypeStruct((N,), jnp.float32),
        jax.ShapeDtypeStruct((N,), jnp.float32),
        jax.ShapeDtypeStruct((N, D), jnp.float32),
    ),
    mesh=_MESH,
    scratch_types=[
        pltpu.VMEM_SHARED((DEG_ROWS,), jnp.float32),
        pltpu.VMEM((BLK,), jnp.int32),
        pltpu.VMEM((1, BLK), jnp.int32),
        pltpu.VMEM((BLK,), jnp.float32),
        pltpu.VMEM((320,), jnp.float32),
        pltpu.VMEM((16,), jnp.float32),
        pltpu.VMEM((16,), jnp.float32),
        pltpu.VMEM((16,), jnp.float32),
        pltpu.VMEM((16, D), jnp.float32),
        pltpu.VMEM((16, D), jnp.float32),
    ],
    compiler_params=_SC_PARAMS,
)
def _deg_kernel(feat_hbm, dst_hbm, norm_hbm, norm2_hbm, t1_hbm,
                deg_sh, dbuf, dloc, ones_b, zbuf, dcopy, nbuf, n2buf,
                hbuf, tbuf):
    _deg_body(feat_hbm, dst_hbm, norm_hbm, norm2_hbm, t1_hbm,
              deg_sh, dbuf, dloc, ones_b, zbuf, dcopy, nbuf, n2buf,
              hbuf, tbuf)


def _hop_body(tbl_hbm, src_hbm, dst_hbm, yp_hbm,
              acc_sh, sbuf, dblk, rowbuf):
    c = lax.axis_index("c")
    s = lax.axis_index("s")

    zero16 = jnp.zeros((16,), jnp.float32)

    def zrow(r, _):
        for q in range(D // 16):
            rowbuf[r, pl.ds(q * 16, 16)] = zero16
        return _

    lax.fori_loop(0, BLK, zrow, 0, unroll=False)
    rows0 = jnp.minimum(s * 640, N - 640)
    for m in range(5):
        pltpu.sync_copy(rowbuf, acc_sh.at[pl.ds(rows0 + m * 128, 128)])
    plsc.subcore_barrier()

    nb = jnp.where(s < HOP_BPC % NS, HOP_BPC // NS + 1, HOP_BPC // NS)

    def blk_body(j, _):
        g = c * HOP_BPC + s + NS * j
        base = g * BLK
        pltpu.sync_copy(src_hbm.at[pl.ds(base, BLK)], sbuf)
        pltpu.sync_copy(dst_hbm.at[pl.ds(base, BLK)], dblk.at[0])
        pltpu.sync_copy(tbl_hbm.at[sbuf], rowbuf)
        pltpu.sync_copy(rowbuf, acc_sh.at[dblk.at[0]], add=True)
        return _

    lax.fori_loop(0, nb, blk_body, 0, unroll=False)
    plsc.subcore_barrier()

    for m in range(5):
        pltpu.sync_copy(acc_sh.at[pl.ds(rows0 + m * 128, 128)], rowbuf)
        pltpu.sync_copy(rowbuf, yp_hbm.at[c, pl.ds(rows0 + m * 128, 128)])


@functools.partial(
    pl.kernel,
    out_type=jax.ShapeDtypeStruct((NC, N, D), jnp.float32),
    mesh=_MESH,
    scratch_types=[
        pltpu.VMEM_SHARED((N, D), jnp.float32),
        pltpu.VMEM((BLK,), jnp.int32),
        pltpu.VMEM((1, BLK), jnp.int32),
        pltpu.VMEM((BLK, D), jnp.float32),
    ],
    compiler_params=_SC_PARAMS,
)
def _hop_kernel(tbl_hbm, src_hbm, dst_hbm, yp_hbm, acc_sh, sbuf, dblk, rowbuf):
    _hop_body(tbl_hbm, src_hbm, dst_hbm, yp_hbm, acc_sh, sbuf, dblk, rowbuf)


_RB = 400


def _scale2_body(ya_ref, yb_ref, s_ref, o_ref):
    o_ref[...] = (ya_ref[...] + yb_ref[...]) * s_ref[...]


def _combine_scale(yp, svec):
    return pl.pallas_call(
        _scale2_body,
        out_shape=jax.ShapeDtypeStruct((N, D), jnp.float32),
        grid=(N // _RB,),
        in_specs=[
            pl.BlockSpec((_RB, D), lambda i: (i, 0)),
            pl.BlockSpec((_RB, D), lambda i: (i, 0)),
            pl.BlockSpec((_RB, 1), lambda i: (i, 0)),
        ],
        out_specs=pl.BlockSpec((_RB, D), lambda i: (i, 0)),
    )(yp[0], yp[1], svec.reshape(N, 1))


def _linear_body(ya_ref, yb_ref, s_ref, w_ref, b_ref, o_ref):
    x = (ya_ref[...] + yb_ref[...]) * s_ref[...]
    o_ref[...] = jnp.dot(x, w_ref[...],
                         preferred_element_type=jnp.float32) + b_ref[...]


def _combine_scale_linear(yp, svec, W, b):
    return pl.pallas_call(
        _linear_body,
        out_shape=jax.ShapeDtypeStruct((N, D), jnp.float32),
        grid=(N // _RB,),
        in_specs=[
            pl.BlockSpec((_RB, D), lambda i: (i, 0)),
            pl.BlockSpec((_RB, D), lambda i: (i, 0)),
            pl.BlockSpec((_RB, 1), lambda i: (i, 0)),
            pl.BlockSpec((D, D), lambda i: (0, 0)),
            pl.BlockSpec((1, D), lambda i: (0, 0)),
        ],
        out_specs=pl.BlockSpec((_RB, D), lambda i: (i, 0)),
    )(yp[0], yp[1], svec.reshape(N, 1), W, b.reshape(1, D))


def kernel(features, edge_index, W, b):
    src = edge_index[0]
    dst = edge_index[1]
    norm, norm2, t1 = _deg_kernel(features, dst)
    y1p = _hop_kernel(t1, src, dst)
    t2 = _combine_scale(y1p, norm2)
    y2p = _hop_kernel(t2, src, dst)
    return _combine_scale_linear(y2p, norm, W, b)


# speedup vs baseline: 9.1133x; 1.9208x over previous
import functools

import jax
import jax.numpy as jnp
from jax import lax
from jax.experimental import pallas as pl
from jax.experimental.pallas import tpu as pltpu
from jax.experimental.pallas import tpu_sc as plsc

N = 10000
E = 320000
D = 128
NC = 2
NS = 16
BLK = 128
NBLK = E // BLK
HOP_BPC = NBLK // NC

LO1 = 5120
DEG_ROWS = 48
TRASH = 5120
DCH = 2000

_MESH = plsc.VectorSubcoreMesh(core_axis_name="c", subcore_axis_name="s")
_SC_PARAMS = pltpu.CompilerParams(needs_layout_passes=False)


def _newton_rsqrt(d):
    i = lax.bitcast_convert_type(d, jnp.int32)
    i = jnp.int32(0x5F3759DF) - (i >> 1)
    y = lax.bitcast_convert_type(i, jnp.float32)
    for _ in range(3):
        y = y * (1.5 - 0.5 * d * y * y)
    return y


def _deg_body(dst_hbm, norm_hbm, norm2_hbm,
              deg_sh, ldeg, ridx, sbig0, sbig1, zrows, ones16b,
              dbuf2, nbuf1k, n2buf1k, semi0, semi1):
    c = lax.axis_index("c")
    s = lax.axis_index("s")
    lo = c * LO1
    hsz = jnp.where(c == 0, LO1, N - LO1)
    sbig = [sbig0, sbig1]
    semi = [semi0, semi1]

    zero16 = jnp.zeros((16,), jnp.float32)
    one16 = jnp.ones((16,), jnp.float32)
    iota16 = lax.iota(jnp.int32, 16)

    def zr(r, _):
        for q in range(8):
            ldeg[r, pl.ds(q * 16, 16)] = zero16
            zrows[r % 8, pl.ds(q * 16, 16)] = zero16
        return _
    lax.fori_loop(0, DEG_ROWS, zr, 0, unroll=False)
    for i in range(3):
        ridx[0, pl.ds(i * 16, 16)] = iota16 + jnp.int32(16 * i)
    ones16b[pl.ds(0, 16)] = one16

    @pl.when(s < 6)
    def _z():
        pltpu.sync_copy(zrows, deg_sh.at[pl.ds(s * 8, 8)])
    plsc.subcore_barrier()

    ebase = s * (E // NS)

    def start_chunk(k, b):
        return pltpu.async_copy(
            dst_hbm.at[pl.ds(ebase + k * DCH, DCH)], sbig[b], semi[b])

    start_chunk(0, 0)

    def chunk_grp(g, _):
        for b in range(2):
            k = 2 * g + b
            pltpu.make_async_copy(
                dst_hbm.at[pl.ds(ebase, DCH)], sbig[b], semi[b]).wait()
            nxt = k + 1

            @pl.when(nxt < E // NS // DCH)
            def _pre():
                start_chunk(nxt, (b + 1) % 2)

            def vec_body(t, _):
                d = sbig[b][pl.ds(t * 16, 16)]
                inh = jnp.logical_and(d >= lo, d < lo + hsz)
                dl = jnp.where(inh, d - lo, jnp.int32(TRASH))
                plsc.addupdate_scatter(ldeg, [dl >> 7, dl & 127], one16)
                return _

            lax.fori_loop(0, DCH // 16, vec_body, 0, unroll=False)
        return _

    lax.fori_loop(0, E // NS // DCH // 2, chunk_grp, 0, unroll=False)

    pltpu.sync_copy(ldeg, deg_sh.at[ridx.at[0]], add=True)
    plsc.subcore_barrier()

    @pl.when(jnp.logical_and(s < 5, jnp.logical_or(c == 0, s < 4)))
    def _p2():
        pltpu.sync_copy(deg_sh.at[pl.ds(s * 8, 8)], dbuf2)

        def body(t, _):
            row = t >> 3
            off = (t & 7) * 16
            d = jnp.maximum(dbuf2[row, pl.ds(off, 16)], 1.0)
            nbuf1k[pl.ds(t * 16, 16)] = _newton_rsqrt(d)
            n2buf1k[pl.ds(t * 16, 16)] = 1.0 / d
            return _

        lax.fori_loop(0, 64, body, 0, unroll=False)
        base = lo + s * 1024
        pltpu.sync_copy(nbuf1k, norm_hbm.at[pl.ds(base, 1024)])
        pltpu.sync_copy(n2buf1k, norm2_hbm.at[pl.ds(base, 1024)])

    @pl.when(jnp.logical_and(c == 1, s == 4))
    def _p2b():
        pltpu.sync_copy(deg_sh.at[pl.ds(32, 8)], dbuf2)

        def body(t, _):
            row = t >> 3
            off = (t & 7) * 16
            d = jnp.maximum(dbuf2[row, pl.ds(off, 16)], 1.0)
            nbuf1k[pl.ds(t * 16, 16)] = _newton_rsqrt(d)
            n2buf1k[pl.ds(t * 16, 16)] = 1.0 / d
            return _

        lax.fori_loop(0, 49, body, 0, unroll=False)
        pltpu.sync_copy(nbuf1k.at[pl.ds(0, 784)],
                        norm_hbm.at[pl.ds(9216, 784)])
        pltpu.sync_copy(n2buf1k.at[pl.ds(0, 784)],
                        norm2_hbm.at[pl.ds(9216, 784)])


@functools.partial(
    pl.kernel,
    out_type=(
        jax.ShapeDtypeStruct((N,), jnp.float32),
        jax.ShapeDtypeStruct((N,), jnp.float32),
    ),
    mesh=_MESH,
    scratch_types=[
        pltpu.VMEM_SHARED((DEG_ROWS, 128), jnp.float32),
        pltpu.VMEM((DEG_ROWS, 128), jnp.float32),
        pltpu.VMEM((1, 48), jnp.int32),
        pltpu.VMEM((DCH,), jnp.int32),
        pltpu.VMEM((DCH,), jnp.int32),
        pltpu.VMEM((8, 128), jnp.float32),
        pltpu.VMEM((16,), jnp.float32),
        pltpu.VMEM((8, 128), jnp.float32),
        pltpu.VMEM((1024,), jnp.float32),
        pltpu.VMEM((1024,), jnp.float32),
        pltpu.SemaphoreType.DMA,
        pltpu.SemaphoreType.DMA,
    ],
    compiler_params=_SC_PARAMS,
)
def _deg_kernel(dst_hbm, norm_hbm, norm2_hbm, *rest):
    _deg_body(dst_hbm, norm_hbm, norm2_hbm, *rest)


def _hop_body(tbl_hbm, src_hbm, dst_hbm, yp_hbm, acc_sh, idxb, dblk,
              rb0, rb1, si0, si1, sg0, sg1, ss0, ss1):
    c = lax.axis_index("c")
    s = lax.axis_index("s")
    rb = [rb0, rb1]
    semi = [si0, si1]
    semg = [sg0, sg1]
    sems = [ss0, ss1]

    zero16 = jnp.zeros((16,), jnp.float32)

    def zrow(r, _):
        for q in range(D // 16):
            rb0[r, pl.ds(q * 16, 16)] = zero16
        return _

    lax.fori_loop(0, BLK, zrow, 0, unroll=False)
    rows0 = jnp.minimum(s * 640, N - 640)
    for m in range(5):
        pltpu.sync_copy(rb0, acc_sh.at[pl.ds(rows0 + m * 128, 128)])
    plsc.subcore_barrier()

    nb = jnp.where(s < HOP_BPC % NS, HOP_BPC // NS + 1, HOP_BPC // NS)

    def idx_start(jj, b):
        g = c * HOP_BPC + s + NS * jj
        pltpu.async_copy(
            src_hbm.at[pl.ds(g * BLK, BLK)], idxb.at[b, 0], semi[b])
        pltpu.async_copy(
            dst_hbm.at[pl.ds(g * BLK, BLK)], idxb.at[b, 1], semi[b])

    for b in range(2):
        @pl.when(b < nb)
        def _pro():
            idx_start(b, b)

    def grp_body(gq, _):
        j0 = 2 * gq
        for b in range(2):
            jj = j0 + b

            @pl.when(jj < nb)
            def _a():
                @pl.when(jj >= 2)
                def _w():
                    pltpu.make_async_copy(
                        rb[b], acc_sh.at[dblk.at[b]], sems[b]).wait()
                pltpu.make_async_copy(
                    src_hbm.at[pl.ds(0, BLK)], idxb.at[b, 0], semi[b]).wait()
                pltpu.make_async_copy(
                    dst_hbm.at[pl.ds(0, BLK)], idxb.at[b, 1], semi[b]).wait()
                pltpu.async_copy(
                    tbl_hbm.at[idxb.at[b, 0]], rb[b], semg[b])
        for b in range(2):
            jj = j0 + b

            @pl.when(jj < nb)
            def _b():
                pltpu.make_async_copy(
                    tbl_hbm.at[idxb.at[b, 0]], rb[b], semg[b]).wait()
                for q in range(BLK // 16):
                    dblk[b, pl.ds(q * 16, 16)] = idxb[b, 1, pl.ds(q * 16, 16)]
                pltpu.async_copy(
                    rb[b], acc_sh.at[dblk.at[b]], sems[b], add=True)

                @pl.when(jj + 2 < nb)
                def _n():
                    idx_start(jj + 2, b)
        return _

    lax.fori_loop(0, 40, grp_body, 0, unroll=False)
    for b in range(2):
        pltpu.make_async_copy(rb[b], acc_sh.at[dblk.at[b]], sems[b]).wait()
    plsc.subcore_barrier()

    for m in range(5):
        pltpu.sync_copy(acc_sh.at[pl.ds(rows0 + m * 128, 128)], rb0)
        pltpu.sync_copy(rb0, yp_hbm.at[c, pl.ds(rows0 + m * 128, 128)])


@functools.partial(
    pl.kernel,
    out_type=jax.ShapeDtypeStruct((NC, N, D), jnp.float32),
    mesh=_MESH,
    scratch_types=[
        pltpu.VMEM_SHARED((N, D), jnp.float32),
        pltpu.VMEM((2, 2, BLK), jnp.int32),
        pltpu.VMEM((2, BLK), jnp.int32),
        pltpu.VMEM((BLK, D), jnp.float32),
        pltpu.VMEM((BLK, D), jnp.float32),
        pltpu.SemaphoreType.DMA, pltpu.SemaphoreType.DMA,
        pltpu.SemaphoreType.DMA, pltpu.SemaphoreType.DMA,
        pltpu.SemaphoreType.DMA, pltpu.SemaphoreType.DMA,
    ],
    compiler_params=_SC_PARAMS,
)
def _hop_kernel(tbl_hbm, src_hbm, dst_hbm, yp_hbm, *rest):
    _hop_body(tbl_hbm, src_hbm, dst_hbm, yp_hbm, *rest)


_RB = 400


def _scale1_body(x_ref, s_ref, o_ref):
    o_ref[...] = x_ref[...] * s_ref[...]


def _scale1(x, svec):
    return pl.pallas_call(
        _scale1_body,
        out_shape=jax.ShapeDtypeStruct((N, D), jnp.float32),
        grid=(N // _RB,),
        in_specs=[
            pl.BlockSpec((_RB, D), lambda i: (i, 0)),
            pl.BlockSpec((_RB, 1), lambda i: (i, 0)),
        ],
        out_specs=pl.BlockSpec((_RB, D), lambda i: (i, 0)),
    )(x, svec.reshape(N, 1))


def _scale2_body(ya_ref, yb_ref, s_ref, o_ref):
    o_ref[...] = (ya_ref[...] + yb_ref[...]) * s_ref[...]


def _combine_scale(yp, svec):
    return pl.pallas_call(
        _scale2_body,
        out_shape=jax.ShapeDtypeStruct((N, D), jnp.float32),
        grid=(N // _RB,),
        in_specs=[
            pl.BlockSpec((_RB, D), lambda i: (i, 0)),
            pl.BlockSpec((_RB, D), lambda i: (i, 0)),
            pl.BlockSpec((_RB, 1), lambda i: (i, 0)),
        ],
        out_specs=pl.BlockSpec((_RB, D), lambda i: (i, 0)),
    )(yp[0], yp[1], svec.reshape(N, 1))


def _linear_body(ya_ref, yb_ref, s_ref, w_ref, b_ref, o_ref):
    x = (ya_ref[...] + yb_ref[...]) * s_ref[...]
    o_ref[...] = jnp.dot(x, w_ref[...],
                         preferred_element_type=jnp.float32) + b_ref[...]


def _combine_scale_linear(yp, svec, W, b):
    return pl.pallas_call(
        _linear_body,
        out_shape=jax.ShapeDtypeStruct((N, D), jnp.float32),
        grid=(N // _RB,),
        in_specs=[
            pl.BlockSpec((_RB, D), lambda i: (i, 0)),
            pl.BlockSpec((_RB, D), lambda i: (i, 0)),
            pl.BlockSpec((_RB, 1), lambda i: (i, 0)),
            pl.BlockSpec((D, D), lambda i: (0, 0)),
            pl.BlockSpec((1, D), lambda i: (0, 0)),
        ],
        out_specs=pl.BlockSpec((_RB, D), lambda i: (i, 0)),
    )(yp[0], yp[1], svec.reshape(N, 1), W, b.reshape(1, D))


def kernel(features, edge_index, W, b):
    src = edge_index[0]
    dst = edge_index[1]
    norm, norm2 = _deg_kernel(dst)
    t1 = _scale1(features, norm)
    y1p = _hop_kernel(t1, src, dst)
    t2 = _combine_scale(y1p, norm2)
    y2p = _hop_kernel(t2, src, dst)
    return _combine_scale_linear(y2p, norm, W, b)


# speedup vs baseline: 9.7455x; 1.0694x over previous
import functools

import jax
import jax.numpy as jnp
from jax import lax
from jax.experimental import pallas as pl
from jax.experimental.pallas import tpu as pltpu
from jax.experimental.pallas import tpu_sc as plsc

N = 10000
E = 320000
D = 128
NC = 2
NS = 16
BLK = 128
NBLK = E // BLK
HOP_BPC = NBLK // NC

LO1 = 5120
DEG_ROWS = 48
TRASH = 5120
DCH = 2000

_MESH = plsc.VectorSubcoreMesh(core_axis_name="c", subcore_axis_name="s")
_SC_PARAMS = pltpu.CompilerParams(needs_layout_passes=False)


def _newton_rsqrt(d):
    i = lax.bitcast_convert_type(d, jnp.int32)
    i = jnp.int32(0x5F3759DF) - (i >> 1)
    y = lax.bitcast_convert_type(i, jnp.float32)
    for _ in range(3):
        y = y * (1.5 - 0.5 * d * y * y)
    return y


def _deg_body(dst_hbm, norm_hbm, norm2_hbm,
              deg_sh, ldeg, ridx, sbig0, sbig1, zrows, ones16b,
              dbuf2, nbuf1k, n2buf1k, semi0, semi1):
    c = lax.axis_index("c")
    s = lax.axis_index("s")
    lo = c * LO1
    hsz = jnp.where(c == 0, LO1, N - LO1)
    sbig = [sbig0, sbig1]
    semi = [semi0, semi1]

    zero16 = jnp.zeros((16,), jnp.float32)
    one16 = jnp.ones((16,), jnp.float32)
    iota16 = lax.iota(jnp.int32, 16)

    def zr(r, _):
        for q in range(8):
            ldeg[r, pl.ds(q * 16, 16)] = zero16
            zrows[r % 8, pl.ds(q * 16, 16)] = zero16
        return _
    lax.fori_loop(0, DEG_ROWS, zr, 0, unroll=False)
    for i in range(3):
        ridx[0, pl.ds(i * 16, 16)] = iota16 + jnp.int32(16 * i)
    ones16b[pl.ds(0, 16)] = one16

    @pl.when(s < 6)
    def _z():
        pltpu.sync_copy(zrows, deg_sh.at[pl.ds(s * 8, 8)])
    plsc.subcore_barrier()

    ebase = s * (E // NS)

    def start_chunk(k, b):
        return pltpu.async_copy(
            dst_hbm.at[pl.ds(ebase + k * DCH, DCH)], sbig[b], semi[b])

    start_chunk(0, 0)

    def chunk_grp(g, _):
        for b in range(2):
            k = 2 * g + b
            pltpu.make_async_copy(
                dst_hbm.at[pl.ds(ebase, DCH)], sbig[b], semi[b]).wait()
            nxt = k + 1

            @pl.when(nxt < E // NS // DCH)
            def _pre():
                start_chunk(nxt, (b + 1) % 2)

            def vec_body(t, _):
                d = sbig[b][pl.ds(t * 16, 16)]
                inh = jnp.logical_and(d >= lo, d < lo + hsz)
                dl = jnp.where(inh, d - lo, jnp.int32(TRASH))
                plsc.addupdate_scatter(ldeg, [dl >> 7, dl & 127], one16)
                return _

            lax.fori_loop(0, DCH // 16, vec_body, 0, unroll=False)
        return _

    lax.fori_loop(0, E // NS // DCH // 2, chunk_grp, 0, unroll=False)

    pltpu.sync_copy(ldeg, deg_sh.at[ridx.at[0]], add=True)
    plsc.subcore_barrier()

    @pl.when(jnp.logical_and(s < 5, jnp.logical_or(c == 0, s < 4)))
    def _p2():
        pltpu.sync_copy(deg_sh.at[pl.ds(s * 8, 8)], dbuf2)

        def body(t, _):
            row = t >> 3
            off = (t & 7) * 16
            d = jnp.maximum(dbuf2[row, pl.ds(off, 16)], 1.0)
            nbuf1k[pl.ds(t * 16, 16)] = _newton_rsqrt(d)
            n2buf1k[pl.ds(t * 16, 16)] = 1.0 / d
            return _

        lax.fori_loop(0, 64, body, 0, unroll=False)
        base = lo + s * 1024
        pltpu.sync_copy(nbuf1k, norm_hbm.at[pl.ds(base, 1024)])
        pltpu.sync_copy(n2buf1k, norm2_hbm.at[pl.ds(base, 1024)])

    @pl.when(jnp.logical_and(c == 1, s == 4))
    def _p2b():
        pltpu.sync_copy(deg_sh.at[pl.ds(32, 8)], dbuf2)

        def body(t, _):
            row = t >> 3
            off = (t & 7) * 16
            d = jnp.maximum(dbuf2[row, pl.ds(off, 16)], 1.0)
            nbuf1k[pl.ds(t * 16, 16)] = _newton_rsqrt(d)
            n2buf1k[pl.ds(t * 16, 16)] = 1.0 / d
            return _

        lax.fori_loop(0, 49, body, 0, unroll=False)
        pltpu.sync_copy(nbuf1k.at[pl.ds(0, 784)],
                        norm_hbm.at[pl.ds(9216, 784)])
        pltpu.sync_copy(n2buf1k.at[pl.ds(0, 784)],
                        norm2_hbm.at[pl.ds(9216, 784)])


@functools.partial(
    pl.kernel,
    out_type=(
        jax.ShapeDtypeStruct((N,), jnp.float32),
        jax.ShapeDtypeStruct((N,), jnp.float32),
    ),
    mesh=_MESH,
    scratch_types=[
        pltpu.VMEM_SHARED((DEG_ROWS, 128), jnp.float32),
        pltpu.VMEM((DEG_ROWS, 128), jnp.float32),
        pltpu.VMEM((1, 48), jnp.int32),
        pltpu.VMEM((DCH,), jnp.int32),
        pltpu.VMEM((DCH,), jnp.int32),
        pltpu.VMEM((8, 128), jnp.float32),
        pltpu.VMEM((16,), jnp.float32),
        pltpu.VMEM((8, 128), jnp.float32),
        pltpu.VMEM((1024,), jnp.float32),
        pltpu.VMEM((1024,), jnp.float32),
        pltpu.SemaphoreType.DMA,
        pltpu.SemaphoreType.DMA,
    ],
    compiler_params=_SC_PARAMS,
)
def _deg_kernel(dst_hbm, norm_hbm, norm2_hbm, *rest):
    _deg_body(dst_hbm, norm_hbm, norm2_hbm, *rest)


def _hop_core(two_inputs, x_hbm, sv_hbm, src_hbm, dst_hbm, tbl_hbm, yp_hbm,
              acc_sh, idxb, dblk, rb0, rb1, rb2, sv640,
              si0, si1, si2, sg0, sg1, sg2, ss0, ss1, ss2, sw0, sw1):
    c = lax.axis_index("c")
    s = lax.axis_index("s")
    rb = [rb0, rb1, rb2]
    semi = [si0, si1, si2]
    semg = [sg0, sg1, sg2]
    sems = [ss0, ss1, ss2]
    semw = [sw0, sw1]
    NSLOT = 3
    cN = c * N

    zero16 = jnp.zeros((16,), jnp.float32)

    def zrow(r, _):
        for q in range(D // 16):
            rb0[r, pl.ds(q * 16, 16)] = zero16
        return _

    lax.fori_loop(0, BLK, zrow, 0, unroll=False)
    rows0 = jnp.minimum(s * 640, N - 640)
    for m in range(5):
        pltpu.sync_copy(rb0, acc_sh.at[pl.ds(rows0 + m * 128, 128)])

    for m in range(5):
        b = m % 2
        r0 = rows0 + m * 128
        if m >= 2:
            pltpu.make_async_copy(
                rb[b], tbl_hbm.at[pl.ds(0, BLK)], semw[b]).wait()
        pltpu.sync_copy(sv_hbm.at[pl.ds(r0, BLK)], sv640)
        if two_inputs:
            pltpu.sync_copy(x_hbm.at[0, pl.ds(r0, BLK)], rb[b])
            pltpu.sync_copy(x_hbm.at[1, pl.ds(r0, BLK)], rb2)
        else:
            pltpu.sync_copy(x_hbm.at[pl.ds(r0, BLK)], rb[b])

        def scl_body(row, _):
            sp = plsc.load_gather(sv640, [jnp.full((16,), row, jnp.int32)])
            for q in range(D // 16):
                v = rb[b][row, pl.ds(q * 16, 16)]
                if two_inputs:
                    v = v + rb2[row, pl.ds(q * 16, 16)]
                rb[b][row, pl.ds(q * 16, 16)] = v * sp
            return _

        lax.fori_loop(0, BLK, scl_body, 0, unroll=False)
        pltpu.async_copy(rb[b], tbl_hbm.at[pl.ds(cN + r0, BLK)], semw[b])
    for b in range(2):
        pltpu.make_async_copy(rb[b], tbl_hbm.at[pl.ds(0, BLK)], semw[b]).wait()
    plsc.subcore_barrier()

    nb = jnp.where(s < HOP_BPC % NS, HOP_BPC // NS + 1, HOP_BPC // NS)

    def idx_start(jj, b):
        g = c * HOP_BPC + s + NS * jj
        pltpu.async_copy(
            src_hbm.at[pl.ds(g * BLK, BLK)], idxb.at[b, 0], semi[b])
        pltpu.async_copy(
            dst_hbm.at[pl.ds(g * BLK, BLK)], idxb.at[b, 1], semi[b])

    for b in range(NSLOT):
        @pl.when(b < nb)
        def _pro():
            idx_start(b, b)

    def grp_body(gq, _):
        j0 = NSLOT * gq
        for b in range(NSLOT):
            jj = j0 + b

            @pl.when(jj < nb)
            def _a():
                @pl.when(jj >= NSLOT)
                def _w():
                    pltpu.make_async_copy(
                        rb[b], acc_sh.at[dblk.at[b]], sems[b]).wait()
                pltpu.make_async_copy(
                    src_hbm.at[pl.ds(0, BLK)], idxb.at[b, 0], semi[b]).wait()
                pltpu.make_async_copy(
                    dst_hbm.at[pl.ds(0, BLK)], idxb.at[b, 1], semi[b]).wait()
                for q in range(BLK // 16):
                    idxb[b, 0, pl.ds(q * 16, 16)] = (
                        idxb[b, 0, pl.ds(q * 16, 16)] + cN)
                    dblk[b, pl.ds(q * 16, 16)] = idxb[b, 1, pl.ds(q * 16, 16)]
                pltpu.async_copy(
                    tbl_hbm.at[idxb.at[b, 0]], rb[b], semg[b])
        for b in range(NSLOT):
            jj = j0 + b

            @pl.when(jj < nb)
            def _b():
                pltpu.make_async_copy(
                    tbl_hbm.at[idxb.at[b, 0]], rb[b], semg[b]).wait()
                pltpu.async_copy(
                    rb[b], acc_sh.at[dblk.at[b]], sems[b], add=True)

                @pl.when(jj + NSLOT < nb)
                def _n():
                    idx_start(jj + NSLOT, b)
        return _

    lax.fori_loop(0, 27, grp_body, 0, unroll=False)
    for b in range(NSLOT):
        pltpu.make_async_copy(rb[b], acc_sh.at[dblk.at[b]], sems[b]).wait()
    plsc.subcore_barrier()

    for m in range(5):
        b = m % 2
        if m >= 2:
            pltpu.make_async_copy(
                rb[b], yp_hbm.at[c, pl.ds(0, BLK)], semw[b]).wait()
        pltpu.sync_copy(acc_sh.at[pl.ds(rows0 + m * 128, 128)], rb[b])
        pltpu.async_copy(
            rb[b], yp_hbm.at[c, pl.ds(rows0 + m * 128, BLK)], semw[b])
    for b in range(2):
        pltpu.make_async_copy(
            rb[b], yp_hbm.at[c, pl.ds(0, BLK)], semw[b]).wait()


_HOP_SCRATCH = [
    pltpu.VMEM_SHARED((N, D), jnp.float32),
    pltpu.VMEM((3, 2, BLK), jnp.int32),
    pltpu.VMEM((3, BLK), jnp.int32),
    pltpu.VMEM((BLK, D), jnp.float32),
    pltpu.VMEM((BLK, D), jnp.float32),
    pltpu.VMEM((BLK, D), jnp.float32),
    pltpu.VMEM((BLK,), jnp.float32),
] + [pltpu.SemaphoreType.DMA] * 11

_HOP_OUT = (
    jax.ShapeDtypeStruct((NC * N, D), jnp.float32),
    jax.ShapeDtypeStruct((NC, N, D), jnp.float32),
)

_hop1_kernel = pl.kernel(
    functools.partial(_hop_core, False),
    out_type=_HOP_OUT, mesh=_MESH, scratch_types=_HOP_SCRATCH,
    compiler_params=_SC_PARAMS)

_hop2_kernel = pl.kernel(
    functools.partial(_hop_core, True),
    out_type=_HOP_OUT, mesh=_MESH, scratch_types=_HOP_SCRATCH,
    compiler_params=_SC_PARAMS)


_RB = 400


def _linear_body(ya_ref, yb_ref, s_ref, w_ref, b_ref, o_ref):
    x = (ya_ref[...] + yb_ref[...]) * s_ref[...]
    o_ref[...] = jnp.dot(x, w_ref[...],
                         preferred_element_type=jnp.float32) + b_ref[...]


def _combine_scale_linear(yp, svec, W, b):
    return pl.pallas_call(
        _linear_body,
        out_shape=jax.ShapeDtypeStruct((N, D), jnp.float32),
        grid=(N // _RB,),
        in_specs=[
            pl.BlockSpec((_RB, D), lambda i: (i, 0)),
            pl.BlockSpec((_RB, D), lambda i: (i, 0)),
            pl.BlockSpec((_RB, 1), lambda i: (i, 0)),
            pl.BlockSpec((D, D), lambda i: (0, 0)),
            pl.BlockSpec((1, D), lambda i: (0, 0)),
        ],
        out_specs=pl.BlockSpec((_RB, D), lambda i: (i, 0)),
    )(yp[0], yp[1], svec.reshape(N, 1), W, b.reshape(1, D))


def kernel(features, edge_index, W, b):
    src = edge_index[0]
    dst = edge_index[1]
    norm, norm2 = _deg_kernel(dst)
    _, y1p = _hop1_kernel(features, norm, src, dst)
    _, y2p = _hop2_kernel(y1p, norm2, src, dst)
    return _combine_scale_linear(y2p, norm, W, b)


# speedup vs baseline: 10.4483x; 1.0721x over previous
import functools

import jax
import jax.numpy as jnp
from jax import lax
from jax.experimental import pallas as pl
from jax.experimental.pallas import tpu as pltpu
from jax.experimental.pallas import tpu_sc as plsc

N = 10000
E = 320000
D = 128
NC = 2
NS = 16
BLK = 128
NBLK = E // BLK
HOP_BPC = NBLK // NC

LO1 = 5120
DEG_ROWS = 48
TRASH = 5120
DCH = 2000

_MESH = plsc.VectorSubcoreMesh(core_axis_name="c", subcore_axis_name="s")
_SC_PARAMS = pltpu.CompilerParams(needs_layout_passes=False)


def _newton_rsqrt(d):
    i = lax.bitcast_convert_type(d, jnp.int32)
    i = jnp.int32(0x5F3759DF) - (i >> 1)
    y = lax.bitcast_convert_type(i, jnp.float32)
    for _ in range(3):
        y = y * (1.5 - 0.5 * d * y * y)
    return y


def _deg_body(dst_hbm, norm_hbm, norm2_hbm,
              deg_sh, ldeg, ridx, sbig0, sbig1, zrows, ones16b,
              dbuf2, nbuf1k, n2buf1k, semi0, semi1):
    c = lax.axis_index("c")
    s = lax.axis_index("s")
    lo = c * LO1
    hsz = jnp.where(c == 0, LO1, N - LO1)
    sbig = [sbig0, sbig1]
    semi = [semi0, semi1]

    zero16 = jnp.zeros((16,), jnp.float32)
    one16 = jnp.ones((16,), jnp.float32)
    iota16 = lax.iota(jnp.int32, 16)

    def zr(r, _):
        for q in range(8):
            ldeg[r, pl.ds(q * 16, 16)] = zero16
            zrows[r % 8, pl.ds(q * 16, 16)] = zero16
        return _
    lax.fori_loop(0, DEG_ROWS, zr, 0, unroll=False)
    for i in range(3):
        ridx[0, pl.ds(i * 16, 16)] = iota16 + jnp.int32(16 * i)
    ones16b[pl.ds(0, 16)] = one16

    @pl.when(s < 6)
    def _z():
        pltpu.sync_copy(zrows, deg_sh.at[pl.ds(s * 8, 8)])
    plsc.subcore_barrier()

    ebase = s * (E // NS)

    def start_chunk(k, b):
        return pltpu.async_copy(
            dst_hbm.at[pl.ds(ebase + k * DCH, DCH)], sbig[b], semi[b])

    start_chunk(0, 0)

    def chunk_grp(g, _):
        for b in range(2):
            k = 2 * g + b
            pltpu.make_async_copy(
                dst_hbm.at[pl.ds(ebase, DCH)], sbig[b], semi[b]).wait()
            nxt = k + 1

            @pl.when(nxt < E // NS // DCH)
            def _pre():
                start_chunk(nxt, (b + 1) % 2)

            def vec_body(t, _):
                d = sbig[b][pl.ds(t * 16, 16)]
                inh = jnp.logical_and(d >= lo, d < lo + hsz)
                dl = jnp.where(inh, d - lo, jnp.int32(TRASH))
                plsc.addupdate_scatter(ldeg, [dl >> 7, dl & 127], one16)
                return _

            lax.fori_loop(0, DCH // 16, vec_body, 0, unroll=False)
        return _

    lax.fori_loop(0, E // NS // DCH // 2, chunk_grp, 0, unroll=False)

    pltpu.sync_copy(ldeg, deg_sh.at[ridx.at[0]], add=True)
    plsc.subcore_barrier()

    @pl.when(jnp.logical_and(s < 5, jnp.logical_or(c == 0, s < 4)))
    def _p2():
        pltpu.sync_copy(deg_sh.at[pl.ds(s * 8, 8)], dbuf2)

        def body(t, _):
            row = t >> 3
            off = (t & 7) * 16
            d = jnp.maximum(dbuf2[row, pl.ds(off, 16)], 1.0)
            nbuf1k[pl.ds(t * 16, 16)] = _newton_rsqrt(d)
            n2buf1k[pl.ds(t * 16, 16)] = 1.0 / d
            return _

        lax.fori_loop(0, 64, body, 0, unroll=False)
        base = lo + s * 1024
        pltpu.sync_copy(nbuf1k, norm_hbm.at[pl.ds(base, 1024)])
        pltpu.sync_copy(n2buf1k, norm2_hbm.at[pl.ds(base, 1024)])

    @pl.when(jnp.logical_and(c == 1, s == 4))
    def _p2b():
        pltpu.sync_copy(deg_sh.at[pl.ds(32, 8)], dbuf2)

        def body(t, _):
            row = t >> 3
            off = (t & 7) * 16
            d = jnp.maximum(dbuf2[row, pl.ds(off, 16)], 1.0)
            nbuf1k[pl.ds(t * 16, 16)] = _newton_rsqrt(d)
            n2buf1k[pl.ds(t * 16, 16)] = 1.0 / d
            return _

        lax.fori_loop(0, 49, body, 0, unroll=False)
        pltpu.sync_copy(nbuf1k.at[pl.ds(0, 784)],
                        norm_hbm.at[pl.ds(9216, 784)])
        pltpu.sync_copy(n2buf1k.at[pl.ds(0, 784)],
                        norm2_hbm.at[pl.ds(9216, 784)])


@functools.partial(
    pl.kernel,
    out_type=(
        jax.ShapeDtypeStruct((N,), jnp.float32),
        jax.ShapeDtypeStruct((N,), jnp.float32),
    ),
    mesh=_MESH,
    scratch_types=[
        pltpu.VMEM_SHARED((DEG_ROWS, 128), jnp.float32),
        pltpu.VMEM((DEG_ROWS, 128), jnp.float32),
        pltpu.VMEM((1, 48), jnp.int32),
        pltpu.VMEM((DCH,), jnp.int32),
        pltpu.VMEM((DCH,), jnp.int32),
        pltpu.VMEM((8, 128), jnp.float32),
        pltpu.VMEM((16,), jnp.float32),
        pltpu.VMEM((8, 128), jnp.float32),
        pltpu.VMEM((1024,), jnp.float32),
        pltpu.VMEM((1024,), jnp.float32),
        pltpu.SemaphoreType.DMA,
        pltpu.SemaphoreType.DMA,
    ],
    compiler_params=_SC_PARAMS,
)
def _deg_kernel(dst_hbm, norm_hbm, norm2_hbm, *rest):
    _deg_body(dst_hbm, norm_hbm, norm2_hbm, *rest)


def _hop_body(tbl_hbm, src_hbm, dst_hbm, yp_hbm, acc_sh, idxb, dblk,
              rb0, rb1, rb2, si0, si1, si2, sg0, sg1, sg2, ss0, ss1, ss2,
              sw0, sw1):
    c = lax.axis_index("c")
    s = lax.axis_index("s")
    rb = [rb0, rb1, rb2]
    semi = [si0, si1, si2]
    semg = [sg0, sg1, sg2]
    sems = [ss0, ss1, ss2]
    semw = [sw0, sw1]
    NSLOT = 3

    zero16 = jnp.zeros((16,), jnp.float32)

    def zrow(r, _):
        for q in range(D // 16):
            rb0[r, pl.ds(q * 16, 16)] = zero16
        return _

    lax.fori_loop(0, BLK, zrow, 0, unroll=False)
    rows0 = jnp.minimum(s * 640, N - 640)
    for m in range(5):
        pltpu.sync_copy(rb0, acc_sh.at[pl.ds(rows0 + m * 128, 128)])
    plsc.subcore_barrier()

    nb = jnp.where(s < HOP_BPC % NS, HOP_BPC // NS + 1, HOP_BPC // NS)

    def idx_start(jj, b):
        g = c * HOP_BPC + s + NS * jj
        pltpu.async_copy(
            src_hbm.at[pl.ds(g * BLK, BLK)], idxb.at[b, 0], semi[b])
        pltpu.async_copy(
            dst_hbm.at[pl.ds(g * BLK, BLK)], idxb.at[b, 1], semi[b])

    for b in range(NSLOT):
        @pl.when(b < nb)
        def _pro():
            idx_start(b, b)

    def grp_body(gq, _):
        j0 = NSLOT * gq
        for b in range(NSLOT):
            jj = j0 + b

            @pl.when(jj < nb)
            def _a():
                @pl.when(jj >= NSLOT)
                def _w():
                    pltpu.make_async_copy(
                        rb[b], acc_sh.at[dblk.at[b]], sems[b]).wait()
                pltpu.make_async_copy(
                    src_hbm.at[pl.ds(0, BLK)], idxb.at[b, 0], semi[b]).wait()
                pltpu.async_copy(
                    tbl_hbm.at[idxb.at[b, 0]], rb[b], semg[b])
                pltpu.make_async_copy(
                    dst_hbm.at[pl.ds(0, BLK)], idxb.at[b, 1], semi[b]).wait()
                for q in range(BLK // 16):
                    dblk[b, pl.ds(q * 16, 16)] = idxb[b, 1, pl.ds(q * 16, 16)]
        for b in range(NSLOT):
            jj = j0 + b

            @pl.when(jj < nb)
            def _b():
                pltpu.make_async_copy(
                    tbl_hbm.at[idxb.at[b, 0]], rb[b], semg[b]).wait()
                pltpu.async_copy(
                    rb[b], acc_sh.at[dblk.at[b]], sems[b], add=True)

                @pl.when(jj + NSLOT < nb)
                def _n():
                    idx_start(jj + NSLOT, b)
        return _

    lax.fori_loop(0, 27, grp_body, 0, unroll=False)
    for b in range(NSLOT):
        pltpu.make_async_copy(rb[b], acc_sh.at[dblk.at[b]], sems[b]).wait()
    plsc.subcore_barrier()

    for m in range(5):
        b = m % 2
        if m >= 2:
            pltpu.make_async_copy(
                rb[b], yp_hbm.at[c, pl.ds(0, BLK)], semw[b]).wait()
        pltpu.sync_copy(acc_sh.at[pl.ds(rows0 + m * 128, 128)], rb[b])
        pltpu.async_copy(
            rb[b], yp_hbm.at[c, pl.ds(rows0 + m * 128, BLK)], semw[b])
    for b in range(2):
        pltpu.make_async_copy(
            rb[b], yp_hbm.at[c, pl.ds(0, BLK)], semw[b]).wait()


_hop_kernel = pl.kernel(
    _hop_body,
    out_type=jax.ShapeDtypeStruct((NC, N, D), jnp.float32),
    mesh=_MESH,
    scratch_types=[
        pltpu.VMEM_SHARED((N, D), jnp.float32),
        pltpu.VMEM((3, 2, BLK), jnp.int32),
        pltpu.VMEM((3, BLK), jnp.int32),
        pltpu.VMEM((BLK, D), jnp.float32),
        pltpu.VMEM((BLK, D), jnp.float32),
        pltpu.VMEM((BLK, D), jnp.float32),
    ] + [pltpu.SemaphoreType.DMA] * 11,
    compiler_params=_SC_PARAMS)


_RBK = 400


def _scale1_body(x_ref, s_ref, o_ref):
    o_ref[...] = x_ref[...] * s_ref[...]


def _scale1(x, svec):
    return pl.pallas_call(
        _scale1_body,
        out_shape=jax.ShapeDtypeStruct((N, D), jnp.float32),
        grid=(N // _RBK,),
        in_specs=[
            pl.BlockSpec((_RBK, D), lambda i: (i, 0)),
            pl.BlockSpec((_RBK, 1), lambda i: (i, 0)),
        ],
        out_specs=pl.BlockSpec((_RBK, D), lambda i: (i, 0)),
    )(x, svec.reshape(N, 1))


def _scale2_body(ya_ref, yb_ref, s_ref, o_ref):
    o_ref[...] = (ya_ref[...] + yb_ref[...]) * s_ref[...]


def _combine_scale(yp, svec):
    return pl.pallas_call(
        _scale2_body,
        out_shape=jax.ShapeDtypeStruct((N, D), jnp.float32),
        grid=(N // _RBK,),
        in_specs=[
            pl.BlockSpec((_RBK, D), lambda i: (i, 0)),
            pl.BlockSpec((_RBK, D), lambda i: (i, 0)),
            pl.BlockSpec((_RBK, 1), lambda i: (i, 0)),
        ],
        out_specs=pl.BlockSpec((_RBK, D), lambda i: (i, 0)),
    )(yp[0], yp[1], svec.reshape(N, 1))


def _linear_body(ya_ref, yb_ref, s_ref, w_ref, b_ref, o_ref):
    x = (ya_ref[...] + yb_ref[...]) * s_ref[...]
    o_ref[...] = jnp.dot(x, w_ref[...],
                         preferred_element_type=jnp.float32) + b_ref[...]


def _combine_scale_linear(yp, svec, W, b):
    return pl.pallas_call(
        _linear_body,
        out_shape=jax.ShapeDtypeStruct((N, D), jnp.float32),
        grid=(N // _RBK,),
        in_specs=[
            pl.BlockSpec((_RBK, D), lambda i: (i, 0)),
            pl.BlockSpec((_RBK, D), lambda i: (i, 0)),
            pl.BlockSpec((_RBK, 1), lambda i: (i, 0)),
            pl.BlockSpec((D, D), lambda i: (0, 0)),
            pl.BlockSpec((1, D), lambda i: (0, 0)),
        ],
        out_specs=pl.BlockSpec((_RBK, D), lambda i: (i, 0)),
    )(yp[0], yp[1], svec.reshape(N, 1), W, b.reshape(1, D))


def kernel(features, edge_index, W, b):
    src = edge_index[0]
    dst = edge_index[1]
    norm, norm2 = _deg_kernel(dst)
    t1 = _scale1(features, norm)
    y1p = _hop_kernel(t1, src, dst)
    t2 = _combine_scale(y1p, norm2)
    y2p = _hop_kernel(t2, src, dst)
    return _combine_scale_linear(y2p, norm, W, b)


# speedup vs baseline: 10.9353x; 1.0466x over previous
import functools

import jax
import jax.numpy as jnp
from jax import lax
from jax.experimental import pallas as pl
from jax.experimental.pallas import tpu as pltpu
from jax.experimental.pallas import tpu_sc as plsc

N = 10000
E = 320000
D = 128
NC = 2
NS = 16
BLK = 64
NBLK = E // BLK
HOP_BPC = NBLK // NC

LO1 = 5120
DEG_ROWS = 48
TRASH = 5120
DCH = 2000

_MESH = plsc.VectorSubcoreMesh(core_axis_name="c", subcore_axis_name="s")
_SC_PARAMS = pltpu.CompilerParams(needs_layout_passes=False)


def _newton_rsqrt(d):
    i = lax.bitcast_convert_type(d, jnp.int32)
    i = jnp.int32(0x5F3759DF) - (i >> 1)
    y = lax.bitcast_convert_type(i, jnp.float32)
    for _ in range(3):
        y = y * (1.5 - 0.5 * d * y * y)
    return y


def _deg_body(dst_hbm, norm_hbm, norm2_hbm,
              deg_sh, ldeg, ridx, sbig0, sbig1, zrows, ones16b,
              dbuf2, nbuf1k, n2buf1k, semi0, semi1):
    c = lax.axis_index("c")
    s = lax.axis_index("s")
    lo = c * LO1
    hsz = jnp.where(c == 0, LO1, N - LO1)
    sbig = [sbig0, sbig1]
    semi = [semi0, semi1]

    zero16 = jnp.zeros((16,), jnp.float32)
    one16 = jnp.ones((16,), jnp.float32)
    iota16 = lax.iota(jnp.int32, 16)

    def zr(r, _):
        for q in range(8):
            ldeg[r, pl.ds(q * 16, 16)] = zero16
            zrows[r % 8, pl.ds(q * 16, 16)] = zero16
        return _
    lax.fori_loop(0, DEG_ROWS, zr, 0, unroll=False)
    for i in range(3):
        ridx[0, pl.ds(i * 16, 16)] = iota16 + jnp.int32(16 * i)
    ones16b[pl.ds(0, 16)] = one16

    @pl.when(s < 6)
    def _z():
        pltpu.sync_copy(zrows, deg_sh.at[pl.ds(s * 8, 8)])
    plsc.subcore_barrier()

    ebase = s * (E // NS)

    def start_chunk(k, b):
        return pltpu.async_copy(
            dst_hbm.at[pl.ds(ebase + k * DCH, DCH)], sbig[b], semi[b])

    start_chunk(0, 0)

    def chunk_grp(g, _):
        for b in range(2):
            k = 2 * g + b
            pltpu.make_async_copy(
                dst_hbm.at[pl.ds(ebase, DCH)], sbig[b], semi[b]).wait()
            nxt = k + 1

            @pl.when(nxt < E // NS // DCH)
            def _pre():
                start_chunk(nxt, (b + 1) % 2)

            def vec_body(t, _):
                d = sbig[b][pl.ds(t * 16, 16)]
                inh = jnp.logical_and(d >= lo, d < lo + hsz)
                dl = jnp.where(inh, d - lo, jnp.int32(TRASH))
                plsc.addupdate_scatter(ldeg, [dl >> 7, dl & 127], one16)
                return _

            lax.fori_loop(0, DCH // 16, vec_body, 0, unroll=False)
        return _

    lax.fori_loop(0, E // NS // DCH // 2, chunk_grp, 0, unroll=False)

    pltpu.sync_copy(ldeg, deg_sh.at[ridx.at[0]], add=True)
    plsc.subcore_barrier()

    @pl.when(jnp.logical_and(s < 5, jnp.logical_or(c == 0, s < 4)))
    def _p2():
        pltpu.sync_copy(deg_sh.at[pl.ds(s * 8, 8)], dbuf2)

        def body(t, _):
            row = t >> 3
            off = (t & 7) * 16
            d = jnp.maximum(dbuf2[row, pl.ds(off, 16)], 1.0)
            nbuf1k[pl.ds(t * 16, 16)] = _newton_rsqrt(d)
            n2buf1k[pl.ds(t * 16, 16)] = 1.0 / d
            return _

        lax.fori_loop(0, 64, body, 0, unroll=False)
        base = lo + s * 1024
        pltpu.sync_copy(nbuf1k, norm_hbm.at[pl.ds(base, 1024)])
        pltpu.sync_copy(n2buf1k, norm2_hbm.at[pl.ds(base, 1024)])

    @pl.when(jnp.logical_and(c == 1, s == 4))
    def _p2b():
        pltpu.sync_copy(deg_sh.at[pl.ds(32, 8)], dbuf2)

        def body(t, _):
            row = t >> 3
            off = (t & 7) * 16
            d = jnp.maximum(dbuf2[row, pl.ds(off, 16)], 1.0)
            nbuf1k[pl.ds(t * 16, 16)] = _newton_rsqrt(d)
            n2buf1k[pl.ds(t * 16, 16)] = 1.0 / d
            return _

        lax.fori_loop(0, 49, body, 0, unroll=False)
        pltpu.sync_copy(nbuf1k.at[pl.ds(0, 784)],
                        norm_hbm.at[pl.ds(9216, 784)])
        pltpu.sync_copy(n2buf1k.at[pl.ds(0, 784)],
                        norm2_hbm.at[pl.ds(9216, 784)])


@functools.partial(
    pl.kernel,
    out_type=(
        jax.ShapeDtypeStruct((N,), jnp.float32),
        jax.ShapeDtypeStruct((N,), jnp.float32),
    ),
    mesh=_MESH,
    scratch_types=[
        pltpu.VMEM_SHARED((DEG_ROWS, 128), jnp.float32),
        pltpu.VMEM((DEG_ROWS, 128), jnp.float32),
        pltpu.VMEM((1, 48), jnp.int32),
        pltpu.VMEM((DCH,), jnp.int32),
        pltpu.VMEM((DCH,), jnp.int32),
        pltpu.VMEM((8, 128), jnp.float32),
        pltpu.VMEM((16,), jnp.float32),
        pltpu.VMEM((8, 128), jnp.float32),
        pltpu.VMEM((1024,), jnp.float32),
        pltpu.VMEM((1024,), jnp.float32),
        pltpu.SemaphoreType.DMA,
        pltpu.SemaphoreType.DMA,
    ],
    compiler_params=_SC_PARAMS,
)
def _deg_kernel(dst_hbm, norm_hbm, norm2_hbm, *rest):
    _deg_body(dst_hbm, norm_hbm, norm2_hbm, *rest)


def _hop_body(tbl_hbm, src_hbm, dst_hbm, yp_hbm, acc_sh, idxb, dblk,
              rb0, rb1, rb2, rb3, rb4,
              si0, si1, si2, si3, si4,
              sg0, sg1, sg2, sg3, sg4,
              ss0, ss1, ss2, ss3, ss4, sw0, sw1):
    c = lax.axis_index("c")
    s = lax.axis_index("s")
    rb = [rb0, rb1, rb2, rb3, rb4]
    semi = [si0, si1, si2, si3, si4]
    semg = [sg0, sg1, sg2, sg3, sg4]
    sems = [ss0, ss1, ss2, ss3, ss4]
    semw = [sw0, sw1]
    NSLOT = 5

    zero16 = jnp.zeros((16,), jnp.float32)

    def zrow(r, _):
        for q in range(D // 16):
            rb0[r, pl.ds(q * 16, 16)] = zero16
        return _

    lax.fori_loop(0, BLK, zrow, 0, unroll=False)
    rows0 = jnp.minimum(s * 640, N - 640)
    for m in range(10):
        pltpu.sync_copy(rb0, acc_sh.at[pl.ds(rows0 + m * 64, 64)])
    plsc.subcore_barrier()

    nb = jnp.where(s < HOP_BPC % NS, HOP_BPC // NS + 1, HOP_BPC // NS)

    def idx_start(jj, b):
        g = c * HOP_BPC + s + NS * jj
        pltpu.async_copy(
            src_hbm.at[pl.ds(g * BLK, BLK)], idxb.at[b, 0], semi[b])
        pltpu.async_copy(
            dst_hbm.at[pl.ds(g * BLK, BLK)], idxb.at[b, 1], semi[b])

    for b in range(NSLOT):
        @pl.when(b < nb)
        def _pro():
            idx_start(b, b)

    def grp_body(gq, _):
        j0 = NSLOT * gq
        for b in range(NSLOT):
            jj = j0 + b

            @pl.when(jj < nb)
            def _a():
                @pl.when(jj >= NSLOT)
                def _w():
                    pltpu.make_async_copy(
                        rb[b], acc_sh.at[dblk.at[b]], sems[b]).wait()
                pltpu.make_async_copy(
                    src_hbm.at[pl.ds(0, BLK)], idxb.at[b, 0], semi[b]).wait()
                pltpu.async_copy(
                    tbl_hbm.at[idxb.at[b, 0]], rb[b], semg[b])
                pltpu.make_async_copy(
                    dst_hbm.at[pl.ds(0, BLK)], idxb.at[b, 1], semi[b]).wait()
                for q in range(BLK // 16):
                    dblk[b, pl.ds(q * 16, 16)] = idxb[b, 1, pl.ds(q * 16, 16)]
        for b in range(NSLOT):
            jj = j0 + b

            @pl.when(jj < nb)
            def _b():
                pltpu.make_async_copy(
                    tbl_hbm.at[idxb.at[b, 0]], rb[b], semg[b]).wait()
                pltpu.async_copy(
                    rb[b], acc_sh.at[dblk.at[b]], sems[b], add=True)

                @pl.when(jj + NSLOT < nb)
                def _n():
                    idx_start(jj + NSLOT, b)
        return _

    lax.fori_loop(0, 32, grp_body, 0, unroll=False)
    for b in range(NSLOT):
        pltpu.make_async_copy(rb[b], acc_sh.at[dblk.at[b]], sems[b]).wait()
    plsc.subcore_barrier()

    for m in range(10):
        b = m % 2
        if m >= 2:
            pltpu.make_async_copy(
                rb[b], yp_hbm.at[c, pl.ds(0, BLK)], semw[b]).wait()
        pltpu.sync_copy(acc_sh.at[pl.ds(rows0 + m * 64, 64)], rb[b])
        pltpu.async_copy(
            rb[b], yp_hbm.at[c, pl.ds(rows0 + m * 64, BLK)], semw[b])
    for b in range(2):
        pltpu.make_async_copy(
            rb[b], yp_hbm.at[c, pl.ds(0, BLK)], semw[b]).wait()


_hop_kernel = pl.kernel(
    _hop_body,
    out_type=jax.ShapeDtypeStruct((NC, N, D), jnp.float32),
    mesh=_MESH,
    scratch_types=[
        pltpu.VMEM_SHARED((N, D), jnp.float32),
        pltpu.VMEM((5, 2, BLK), jnp.int32),
        pltpu.VMEM((5, BLK), jnp.int32),
    ] + [pltpu.VMEM((BLK, D), jnp.float32)] * 5 \
      + [pltpu.SemaphoreType.DMA] * 17,
    compiler_params=_SC_PARAMS)


_RBK = 400


def _scale1_body(x_ref, s_ref, o_ref):
    o_ref[...] = x_ref[...] * s_ref[...]


def _scale1(x, svec):
    return pl.pallas_call(
        _scale1_body,
        out_shape=jax.ShapeDtypeStruct((N, D), jnp.float32),
        grid=(N // _RBK,),
        in_specs=[
            pl.BlockSpec((_RBK, D), lambda i: (i, 0)),
            pl.BlockSpec((_RBK, 1), lambda i: (i, 0)),
        ],
        out_specs=pl.BlockSpec((_RBK, D), lambda i: (i, 0)),
    )(x, svec.reshape(N, 1))


def _scale2_body(ya_ref, yb_ref, s_ref, o_ref):
    o_ref[...] = (ya_ref[...] + yb_ref[...]) * s_ref[...]


def _combine_scale(yp, svec):
    return pl.pallas_call(
        _scale2_body,
        out_shape=jax.ShapeDtypeStruct((N, D), jnp.float32),
        grid=(N // _RBK,),
        in_specs=[
            pl.BlockSpec((_RBK, D), lambda i: (i, 0)),
            pl.BlockSpec((_RBK, D), lambda i: (i, 0)),
            pl.BlockSpec((_RBK, 1), lambda i: (i, 0)),
        ],
        out_specs=pl.BlockSpec((_RBK, D), lambda i: (i, 0)),
    )(yp[0], yp[1], svec.reshape(N, 1))


def _linear_body(ya_ref, yb_ref, s_ref, w_ref, b_ref, o_ref):
    x = (ya_ref[...] + yb_ref[...]) * s_ref[...]
    o_ref[...] = jnp.dot(x, w_ref[...],
                         preferred_element_type=jnp.float32) + b_ref[...]


def _combine_scale_linear(yp, svec, W, b):
    return pl.pallas_call(
        _linear_body,
        out_shape=jax.ShapeDtypeStruct((N, D), jnp.float32),
        grid=(N // _RBK,),
        in_specs=[
            pl.BlockSpec((_RBK, D), lambda i: (i, 0)),
            pl.BlockSpec((_RBK, D), lambda i: (i, 0)),
            pl.BlockSpec((_RBK, 1), lambda i: (i, 0)),
            pl.BlockSpec((D, D), lambda i: (0, 0)),
            pl.BlockSpec((1, D), lambda i: (0, 0)),
        ],
        out_specs=pl.BlockSpec((_RBK, D), lambda i: (i, 0)),
    )(yp[0], yp[1], svec.reshape(N, 1), W, b.reshape(1, D))


def kernel(features, edge_index, W, b):
    src = edge_index[0]
    dst = edge_index[1]
    norm, norm2 = _deg_kernel(dst)
    t1 = _scale1(features, norm)
    y1p = _hop_kernel(t1, src, dst)
    t2 = _combine_scale(y1p, norm2)
    y2p = _hop_kernel(t2, src, dst)
    return _combine_scale_linear(y2p, norm, W, b)
